# baseline (speedup 1.0000x reference)
import jax
import jax.numpy as jnp
from jax.experimental import pallas as pl
from jax.experimental.pallas import tpu as pltpu

_ROWS = 512


def _copy_body(w_ref, o_ref):
    x = w_ref[...]
    o_ref[0] = x
    o_ref[1] = x


def kernel(tokens, W_pos):
    batch, pos = tokens.shape
    n_ctx, d = W_pos.shape
    grid = (pos // _ROWS,)
    out = pl.pallas_call(
        _copy_body,
        grid=grid,
        in_specs=[pl.BlockSpec((_ROWS, d), lambda i: (i, 0))],
        out_specs=pl.BlockSpec((batch, _ROWS, d), lambda i: (0, i, 0)),
        out_shape=jax.ShapeDtypeStruct((batch, pos, d), W_pos.dtype),
    )(W_pos)
    return out


# speedup vs baseline: 1.0506x; 1.0506x over previous
import jax
import jax.numpy as jnp
from jax.experimental import pallas as pl
from jax.experimental.pallas import tpu as pltpu

_CHUNKS = 8


def _make_body(batch, pos, d):
    rows = pos // _CHUNKS

    def body(w_hbm, o_hbm, vmem, sem_in, sem_out):
        ins = []
        for i in range(_CHUNKS):
            c = pltpu.make_async_copy(
                w_hbm.at[pl.ds(i * rows, rows), :],
                vmem.at[pl.ds(i * rows, rows), :],
                sem_in.at[i],
            )
            c.start()
            ins.append(c)
        outs = []
        for i in range(_CHUNKS):
            ins[i].wait()
            for b in range(batch):
                c = pltpu.make_async_copy(
                    vmem.at[pl.ds(i * rows, rows), :],
                    o_hbm.at[b, pl.ds(i * rows, rows), :],
                    sem_out.at[i, b],
                )
                c.start()
                outs.append(c)
        for c in outs:
            c.wait()

    return body


def kernel(tokens, W_pos):
    batch, pos = tokens.shape
    n_ctx, d = W_pos.shape
    out = pl.pallas_call(
        _make_body(batch, pos, d),
        in_specs=[pl.BlockSpec(memory_space=pl.ANY)],
        out_specs=pl.BlockSpec(memory_space=pl.ANY),
        out_shape=jax.ShapeDtypeStruct((batch, pos, d), W_pos.dtype),
        scratch_shapes=[
            pltpu.VMEM((pos, d), W_pos.dtype),
            pltpu.SemaphoreType.DMA((_CHUNKS,)),
            pltpu.SemaphoreType.DMA((_CHUNKS, 2)),
        ],
    )(W_pos)
    return out
